# Initial kernel scaffold; baseline (speedup 1.0000x reference)
import jax
import jax.numpy as jnp
from jax.experimental import pallas as pl


def kernel(p, x, o, f, down_mask, W, gamma, beta):
    raise NotImplementedError("write your pallas kernel here")


# speedup vs baseline: 11.5169x; 11.5169x over previous
import functools

import jax
import jax.numpy as jnp
from jax import lax
from jax.experimental import pallas as pl
from jax.experimental.pallas import tpu as pltpu
from jax.experimental.pallas import tpu_sc as plsc

_N = 16384
_M = _N // 4
_K = 16
_CIN = 64
_COUT = 64
_FD = 8
_EPS = 1e-5

_R = 128
_QB = 128
_BIGI = 2**30


def _fps_body(px_ref, py_ref, pz_ref, idx_ref, nx_ref, ny_ref, nz_ref):
    row = lax.broadcasted_iota(jnp.int32, (_R, _R), 0)
    col = lax.broadcasted_iota(jnp.int32, (_R, _R), 1)
    iota2 = row * _R + col

    px = px_ref[...]
    py = py_ref[...]
    pz = pz_ref[...]

    idx_ref[0:1, :] = jnp.zeros((1, 1), jnp.int32)
    nx_ref[0:1, :] = px[0:1, 0:1]
    ny_ref[0:1, :] = py[0:1, 0:1]
    nz_ref[0:1, :] = pz[0:1, 0:1]

    def body(i, carry):
        dists, lx, ly, lz = carry
        dx = px - lx
        t = dx * dx
        dy = py - ly
        t = t + dy * dy
        dz = pz - lz
        t = t + dz * dz
        dists = jnp.minimum(dists, t)
        m = jnp.max(dists)
        j = jnp.min(jnp.where(dists == m, iota2, _BIGI))
        idx_ref[pl.ds(i, 1), :] = jnp.reshape(j, (1, 1))
        eq = iota2 == j
        zero = jnp.zeros((), jnp.float32)
        lx = jnp.reshape(jnp.sum(jnp.where(eq, px, zero)), (1, 1))
        ly = jnp.reshape(jnp.sum(jnp.where(eq, py, zero)), (1, 1))
        lz = jnp.reshape(jnp.sum(jnp.where(eq, pz, zero)), (1, 1))
        nx_ref[pl.ds(i, 1), :] = lx
        ny_ref[pl.ds(i, 1), :] = ly
        nz_ref[pl.ds(i, 1), :] = lz
        return dists, lx, ly, lz

    dists0 = jnp.full((_R, _R), 1e10, jnp.float32)
    lax.fori_loop(1, _M, body,
                  (dists0, px[0:1, 0:1], py[0:1, 0:1], pz[0:1, 0:1]))


def _run_fps(px2, py2, pz2):
    out = pl.pallas_call(
        _fps_body,
        out_shape=(
            jax.ShapeDtypeStruct((_M, 1), jnp.int32),
            jax.ShapeDtypeStruct((_M, 1), jnp.float32),
            jax.ShapeDtypeStruct((_M, 1), jnp.float32),
            jax.ShapeDtypeStruct((_M, 1), jnp.float32),
        ),
    )(px2, py2, pz2)
    return out


def _knn_body(nx_ref, ny_ref, nz_ref, pxt_ref, pyt_ref, pzt_ref,
              knn_ref, dmat_ref):
    qx = nx_ref[...]
    qy = ny_ref[...]
    qz = nz_ref[...]
    dx = qx - pxt_ref[...]
    d = dx * dx
    dy = qy - pyt_ref[...]
    d = d + dy * dy
    dz = qz - pzt_ref[...]
    d = d + dz * dz
    col = lax.broadcasted_iota(jnp.int32, (_QB, _N), 1)
    inf = jnp.float32(jnp.inf)
    dmat_ref[...] = d
    for r in range(_K):
        d = dmat_ref[...]
        m = jnp.min(d, axis=1, keepdims=True)
        j = jnp.min(jnp.where(d == m, col, _BIGI), axis=1, keepdims=True)
        knn_ref[:, r:r + 1] = j
        dmat_ref[...] = jnp.where(col == j, inf, d)


def _run_knn(nx, ny, nz, pxt, pyt, pzt):
    grid = _M // _QB
    qspec = pl.BlockSpec((_QB, 1), lambda i: (i, 0))
    pspec = pl.BlockSpec((1, _N), lambda i: (0, 0))
    return pl.pallas_call(
        _knn_body,
        grid=(grid,),
        in_specs=[qspec, qspec, qspec, pspec, pspec, pspec],
        out_specs=pl.BlockSpec((_QB, _K), lambda i: (i, 0)),
        out_shape=jax.ShapeDtypeStruct((_M, _K), jnp.int32),
        scratch_shapes=[pltpu.VMEM((_QB, _N), jnp.float32)],
    )(nx, ny, nz, pxt, pyt, pzt)


def _mm_body(a_ref, b_ref, o_ref):
    o_ref[...] = jnp.dot(a_ref[...], b_ref[...],
                         preferred_element_type=jnp.float32)


def _run_mm(a, b):
    rows, kdim = a.shape
    cols = b.shape[1]
    blk = 1024
    return pl.pallas_call(
        _mm_body,
        grid=(rows // blk,),
        in_specs=[pl.BlockSpec((blk, kdim), lambda i: (i, 0)),
                  pl.BlockSpec((kdim, cols), lambda i: (0, 0))],
        out_specs=pl.BlockSpec((blk, cols), lambda i: (i, 0)),
        out_shape=jax.ShapeDtypeStruct((rows, cols), jnp.float32),
    )(a, b)


_SC_CHUNK = 128


def _run_sc_gather(u, f16, knn_flat, idx_flat):
    nc, ns = 2, 16
    nw = nc * ns
    b_u = (_M * _K) // nw
    n_chunks = b_u // _SC_CHUNK
    b_f = _M // nw

    mesh = plsc.VectorSubcoreMesh(core_axis_name="c", subcore_axis_name="s")

    @functools.partial(
        pl.kernel,
        mesh=mesh,
        compiler_params=pltpu.CompilerParams(use_tc_tiling_on_sc=False),
        out_type=(
            jax.ShapeDtypeStruct((_M * _K, _COUT), jnp.float32),
            jax.ShapeDtypeStruct((_M, 16), jnp.float32),
        ),
        scratch_types=[
            pltpu.VMEM((_SC_CHUNK,), jnp.int32),
            pltpu.VMEM((_SC_CHUNK, _COUT), jnp.float32),
            pltpu.VMEM((b_f,), jnp.int32),
            pltpu.VMEM((b_f, 16), jnp.float32),
            pltpu.SemaphoreType.DMA,
        ],
    )
    def gather_k(u_hbm, f16_hbm, knn_hbm, idx_hbm, gu_hbm, nf_hbm,
                 idxv, rowsv, idxf, rowsf, sem):
        wid = lax.axis_index("s") * nc + lax.axis_index("c")

        def chunk(ci, _):
            base = wid * b_u + ci * _SC_CHUNK
            pltpu.sync_copy(knn_hbm.at[pl.ds(base, _SC_CHUNK)], idxv)
            pltpu.async_copy(u_hbm.at[idxv], rowsv, sem).wait()
            pltpu.sync_copy(rowsv, gu_hbm.at[pl.ds(base, _SC_CHUNK)])
            return 0

        lax.fori_loop(0, n_chunks, chunk, 0)

        fbase = wid * b_f
        pltpu.sync_copy(idx_hbm.at[pl.ds(fbase, b_f)], idxf)
        pltpu.async_copy(f16_hbm.at[idxf], rowsf, sem).wait()
        pltpu.sync_copy(rowsf, nf_hbm.at[pl.ds(fbase, b_f)])

    return gather_k(u, f16, knn_flat, idx_flat)


def _red_body(gu_ref, mx_ref, s1_ref, s2_ref):
    k = pl.program_id(0)
    blk = gu_ref[...]

    @pl.when(k == 0)
    def _init():
        mx_ref[...] = blk
        s1_ref[...] = blk
        s2_ref[...] = blk * blk

    @pl.when(k > 0)
    def _acc():
        mx_ref[...] = jnp.maximum(mx_ref[...], blk)
        s1_ref[...] = s1_ref[...] + blk
        s2_ref[...] = s2_ref[...] + blk * blk


def _run_reduce(gu):
    ospec = pl.BlockSpec((_M, _COUT), lambda k: (0, 0))
    return pl.pallas_call(
        _red_body,
        grid=(_K,),
        in_specs=[pl.BlockSpec((_M, _COUT), lambda k: (k, 0))],
        out_specs=(ospec, ospec, ospec),
        out_shape=(
            jax.ShapeDtypeStruct((_M, _COUT), jnp.float32),
            jax.ShapeDtypeStruct((_M, _COUT), jnp.float32),
            jax.ShapeDtypeStruct((_M, _COUT), jnp.float32),
        ),
    )(gu)


def _fin_body(mx_ref, s1_ref, s2_ref, zc_ref, g_ref, b_ref, o_ref):
    s1 = s1_ref[...]
    s2 = s2_ref[...]
    zc = zc_ref[...]
    kf = jnp.float32(_K)
    tot = jnp.float32(_M * _K)
    csum1 = jnp.sum(s1 - kf * zc, axis=0, keepdims=True)
    csum2 = jnp.sum(s2 - 2.0 * zc * s1 + kf * zc * zc, axis=0, keepdims=True)
    mean = csum1 / tot
    var = csum2 / tot - mean * mean
    inv = lax.rsqrt(var + jnp.float32(_EPS))
    h = (mx_ref[...] - zc - mean) * (inv * g_ref[...]) + b_ref[...]
    o_ref[...] = jnp.maximum(h, jnp.float32(0.0))


def _run_finalize(mx, s1, s2, zc, gamma2, beta2):
    return pl.pallas_call(
        _fin_body,
        out_shape=jax.ShapeDtypeStruct((_M, _COUT), jnp.float32),
    )(mx, s1, s2, zc, gamma2, beta2)


def kernel(p, x, o, f, down_mask, W, gamma, beta):
    del down_mask
    px2 = p[:, 0].reshape(_R, _R)
    py2 = p[:, 1].reshape(_R, _R)
    pz2 = p[:, 2].reshape(_R, _R)

    idx2, nx, ny, nz = _run_fps(px2, py2, pz2)

    pxt = p[:, 0].reshape(1, _N)
    pyt = p[:, 1].reshape(1, _N)
    pzt = p[:, 2].reshape(1, _N)
    knn = _run_knn(nx, ny, nz, pxt, pyt, pzt)

    kdim = 3 + _CIN
    pad = 128 - kdim
    a_all = jnp.concatenate([p, x], axis=1)
    n_p = jnp.concatenate([nx, ny, nz], axis=1)
    a_ctr = jnp.concatenate([n_p, jnp.zeros((_M, _CIN), jnp.float32)], axis=1)
    a = jnp.concatenate([a_all, a_ctr], axis=0)
    a = jnp.concatenate([a, jnp.zeros((a.shape[0], pad), jnp.float32)], axis=1)
    wt = jnp.concatenate([W.T, jnp.zeros((pad, _COUT), jnp.float32)], axis=0)
    uz = _run_mm(a, wt)
    u = uz[:_N]
    zc = uz[_N:]

    knn_flat = knn.T.reshape(-1)
    idx_flat = idx2[:, 0]
    f16 = jnp.concatenate([f, jnp.zeros((_N, 16 - _FD), jnp.float32)], axis=1)
    gu, nf16 = _run_sc_gather(u, f16, knn_flat, idx_flat)

    mx, s1, s2 = _run_reduce(gu)
    out = _run_finalize(mx, s1, s2, zc,
                        gamma.reshape(1, _COUT), beta.reshape(1, _COUT))

    n_f = nf16[:, :_FD]
    n_o = jnp.array([_M], dtype=jnp.int32)
    return (n_p, out, n_o, idx_flat, n_f)


# speedup vs baseline: 11.6635x; 1.0127x over previous
import functools

import jax
import jax.numpy as jnp
from jax import lax
from jax.experimental import pallas as pl
from jax.experimental.pallas import tpu as pltpu
from jax.experimental.pallas import tpu_sc as plsc

_N = 16384
_M = _N // 4
_K = 16
_CIN = 64
_COUT = 64
_FD = 8
_EPS = 1e-5

_R = 128
_QB = 128
_BIGI = 2**30


def _fps_body(px_ref, py_ref, pz_ref, idx_ref, nx_ref, ny_ref, nz_ref):
    row = lax.broadcasted_iota(jnp.int32, (_R, _R), 0)
    col = lax.broadcasted_iota(jnp.int32, (_R, _R), 1)
    iota2 = row * _R + col
    lane1 = lax.broadcasted_iota(jnp.int32, (1, _R), 1)

    px = px_ref[...]
    py = py_ref[...]
    pz = pz_ref[...]

    idx_ref[0:1, :] = jnp.zeros((1, 1), jnp.int32)
    nx_ref[0:1, :] = px[0:1, 0:1]
    ny_ref[0:1, :] = py[0:1, 0:1]
    nz_ref[0:1, :] = pz[0:1, 0:1]

    def body(i, carry):
        dists, lx, ly, lz = carry
        dx = px - lx
        t = dx * dx
        dy = py - ly
        t = t + dy * dy
        dz = pz - lz
        t = t + dz * dz
        dists = jnp.minimum(dists, t)
        m = jnp.max(dists)
        j = jnp.min(jnp.where(dists == m, iota2, _BIGI))
        idx_ref[pl.ds(i, 1), :] = jnp.reshape(j, (1, 1))
        jrow = j // _R
        eq = lane1 == (j - jrow * _R)
        zero = jnp.zeros((), jnp.float32)
        lx = jnp.reshape(
            jnp.sum(jnp.where(eq, px_ref[pl.ds(jrow, 1), :], zero)), (1, 1))
        ly = jnp.reshape(
            jnp.sum(jnp.where(eq, py_ref[pl.ds(jrow, 1), :], zero)), (1, 1))
        lz = jnp.reshape(
            jnp.sum(jnp.where(eq, pz_ref[pl.ds(jrow, 1), :], zero)), (1, 1))
        nx_ref[pl.ds(i, 1), :] = lx
        ny_ref[pl.ds(i, 1), :] = ly
        nz_ref[pl.ds(i, 1), :] = lz
        return dists, lx, ly, lz

    dists0 = jnp.full((_R, _R), 1e10, jnp.float32)
    lax.fori_loop(1, _M, body,
                  (dists0, px[0:1, 0:1], py[0:1, 0:1], pz[0:1, 0:1]))


def _run_fps(px2, py2, pz2):
    out = pl.pallas_call(
        _fps_body,
        out_shape=(
            jax.ShapeDtypeStruct((_M, 1), jnp.int32),
            jax.ShapeDtypeStruct((_M, 1), jnp.float32),
            jax.ShapeDtypeStruct((_M, 1), jnp.float32),
            jax.ShapeDtypeStruct((_M, 1), jnp.float32),
        ),
    )(px2, py2, pz2)
    return out


def _knn_body(nx_ref, ny_ref, nz_ref, pxt_ref, pyt_ref, pzt_ref,
              knn_ref, dmat_ref):
    qx = nx_ref[...]
    qy = ny_ref[...]
    qz = nz_ref[...]
    dx = qx - pxt_ref[...]
    d = dx * dx
    dy = qy - pyt_ref[...]
    d = d + dy * dy
    dz = qz - pzt_ref[...]
    d = d + dz * dz
    col = lax.broadcasted_iota(jnp.int32, (_QB, _N), 1)
    inf = jnp.float32(jnp.inf)
    dmat_ref[...] = d
    for r in range(_K):
        d = dmat_ref[...]
        m = jnp.min(d, axis=1, keepdims=True)
        j = jnp.min(jnp.where(d == m, col, _BIGI), axis=1, keepdims=True)
        knn_ref[:, r:r + 1] = j
        dmat_ref[...] = jnp.where(col == j, inf, d)


def _run_knn(nx, ny, nz, pxt, pyt, pzt):
    grid = _M // _QB
    qspec = pl.BlockSpec((_QB, 1), lambda i: (i, 0))
    pspec = pl.BlockSpec((1, _N), lambda i: (0, 0))
    return pl.pallas_call(
        _knn_body,
        grid=(grid,),
        in_specs=[qspec, qspec, qspec, pspec, pspec, pspec],
        out_specs=pl.BlockSpec((_QB, _K), lambda i: (i, 0)),
        out_shape=jax.ShapeDtypeStruct((_M, _K), jnp.int32),
        scratch_shapes=[pltpu.VMEM((_QB, _N), jnp.float32)],
    )(nx, ny, nz, pxt, pyt, pzt)


def _mm_body(a_ref, b_ref, o_ref):
    o_ref[...] = jnp.dot(a_ref[...], b_ref[...],
                         preferred_element_type=jnp.float32)


def _run_mm(a, b):
    rows, kdim = a.shape
    cols = b.shape[1]
    blk = 1024
    return pl.pallas_call(
        _mm_body,
        grid=(rows // blk,),
        in_specs=[pl.BlockSpec((blk, kdim), lambda i: (i, 0)),
                  pl.BlockSpec((kdim, cols), lambda i: (0, 0))],
        out_specs=pl.BlockSpec((blk, cols), lambda i: (i, 0)),
        out_shape=jax.ShapeDtypeStruct((rows, cols), jnp.float32),
    )(a, b)


_SC_CHUNK = 128


def _run_sc_gather(u, f16, knn_flat, idx_flat):
    nc, ns = 2, 16
    nw = nc * ns
    b_u = (_M * _K) // nw
    n_chunks = b_u // _SC_CHUNK
    b_f = _M // nw

    mesh = plsc.VectorSubcoreMesh(core_axis_name="c", subcore_axis_name="s")

    @functools.partial(
        pl.kernel,
        mesh=mesh,
        compiler_params=pltpu.CompilerParams(use_tc_tiling_on_sc=False),
        out_type=(
            jax.ShapeDtypeStruct((_M * _K, _COUT), jnp.float32),
            jax.ShapeDtypeStruct((_M, 16), jnp.float32),
        ),
        scratch_types=[
            pltpu.VMEM((_SC_CHUNK,), jnp.int32),
            pltpu.VMEM((_SC_CHUNK, _COUT), jnp.float32),
            pltpu.VMEM((b_f,), jnp.int32),
            pltpu.VMEM((b_f, 16), jnp.float32),
            pltpu.SemaphoreType.DMA,
        ],
    )
    def gather_k(u_hbm, f16_hbm, knn_hbm, idx_hbm, gu_hbm, nf_hbm,
                 idxv, rowsv, idxf, rowsf, sem):
        wid = lax.axis_index("s") * nc + lax.axis_index("c")

        def chunk(ci, _):
            base = wid * b_u + ci * _SC_CHUNK
            pltpu.sync_copy(knn_hbm.at[pl.ds(base, _SC_CHUNK)], idxv)
            pltpu.async_copy(u_hbm.at[idxv], rowsv, sem).wait()
            pltpu.sync_copy(rowsv, gu_hbm.at[pl.ds(base, _SC_CHUNK)])
            return 0

        lax.fori_loop(0, n_chunks, chunk, 0)

        fbase = wid * b_f
        pltpu.sync_copy(idx_hbm.at[pl.ds(fbase, b_f)], idxf)
        pltpu.async_copy(f16_hbm.at[idxf], rowsf, sem).wait()
        pltpu.sync_copy(rowsf, nf_hbm.at[pl.ds(fbase, b_f)])

    return gather_k(u, f16, knn_flat, idx_flat)


def _red_body(gu_ref, mx_ref, s1_ref, s2_ref):
    k = pl.program_id(0)
    blk = gu_ref[...]

    @pl.when(k == 0)
    def _init():
        mx_ref[...] = blk
        s1_ref[...] = blk
        s2_ref[...] = blk * blk

    @pl.when(k > 0)
    def _acc():
        mx_ref[...] = jnp.maximum(mx_ref[...], blk)
        s1_ref[...] = s1_ref[...] + blk
        s2_ref[...] = s2_ref[...] + blk * blk


def _run_reduce(gu):
    ospec = pl.BlockSpec((_M, _COUT), lambda k: (0, 0))
    return pl.pallas_call(
        _red_body,
        grid=(_K,),
        in_specs=[pl.BlockSpec((_M, _COUT), lambda k: (k, 0))],
        out_specs=(ospec, ospec, ospec),
        out_shape=(
            jax.ShapeDtypeStruct((_M, _COUT), jnp.float32),
            jax.ShapeDtypeStruct((_M, _COUT), jnp.float32),
            jax.ShapeDtypeStruct((_M, _COUT), jnp.float32),
        ),
    )(gu)


def _fin_body(mx_ref, s1_ref, s2_ref, zc_ref, g_ref, b_ref, o_ref):
    s1 = s1_ref[...]
    s2 = s2_ref[...]
    zc = zc_ref[...]
    kf = jnp.float32(_K)
    tot = jnp.float32(_M * _K)
    csum1 = jnp.sum(s1 - kf * zc, axis=0, keepdims=True)
    csum2 = jnp.sum(s2 - 2.0 * zc * s1 + kf * zc * zc, axis=0, keepdims=True)
    mean = csum1 / tot
    var = csum2 / tot - mean * mean
    inv = lax.rsqrt(var + jnp.float32(_EPS))
    h = (mx_ref[...] - zc - mean) * (inv * g_ref[...]) + b_ref[...]
    o_ref[...] = jnp.maximum(h, jnp.float32(0.0))


def _run_finalize(mx, s1, s2, zc, gamma2, beta2):
    return pl.pallas_call(
        _fin_body,
        out_shape=jax.ShapeDtypeStruct((_M, _COUT), jnp.float32),
    )(mx, s1, s2, zc, gamma2, beta2)


def kernel(p, x, o, f, down_mask, W, gamma, beta):
    del down_mask
    px2 = p[:, 0].reshape(_R, _R)
    py2 = p[:, 1].reshape(_R, _R)
    pz2 = p[:, 2].reshape(_R, _R)

    idx2, nx, ny, nz = _run_fps(px2, py2, pz2)

    pxt = p[:, 0].reshape(1, _N)
    pyt = p[:, 1].reshape(1, _N)
    pzt = p[:, 2].reshape(1, _N)
    knn = _run_knn(nx, ny, nz, pxt, pyt, pzt)

    kdim = 3 + _CIN
    pad = 128 - kdim
    a_all = jnp.concatenate([p, x], axis=1)
    n_p = jnp.concatenate([nx, ny, nz], axis=1)
    a_ctr = jnp.concatenate([n_p, jnp.zeros((_M, _CIN), jnp.float32)], axis=1)
    a = jnp.concatenate([a_all, a_ctr], axis=0)
    a = jnp.concatenate([a, jnp.zeros((a.shape[0], pad), jnp.float32)], axis=1)
    wt = jnp.concatenate([W.T, jnp.zeros((pad, _COUT), jnp.float32)], axis=0)
    uz = _run_mm(a, wt)
    u = uz[:_N]
    zc = uz[_N:]

    knn_flat = knn.T.reshape(-1)
    idx_flat = idx2[:, 0]
    f16 = jnp.concatenate([f, jnp.zeros((_N, 16 - _FD), jnp.float32)], axis=1)
    gu, nf16 = _run_sc_gather(u, f16, knn_flat, idx_flat)

    mx, s1, s2 = _run_reduce(gu)
    out = _run_finalize(mx, s1, s2, zc,
                        gamma.reshape(1, _COUT), beta.reshape(1, _COUT))

    n_f = nf16[:, :_FD]
    n_o = jnp.array([_M], dtype=jnp.int32)
    return (n_p, out, n_o, idx_flat, n_f)
